# baseline (speedup 1.0000x reference)
import functools

import jax
import jax.numpy as jnp
from jax import lax
from jax.experimental import pallas as pl
from jax.experimental.pallas import tpu as pltpu
from jax.experimental.pallas import tpu_sc as plsc

_NS = 16
_L = 16

_NEG = -3.0e38
_POS = 3.0e38

_DN1 = (((1,), (1,)), ((), ()))


def _mm(x, w_ref):
    return lax.dot_general(x, w_ref[...], _DN1,
                           preferred_element_type=jnp.float32)


def _main_body(x_ref, w1_hbm, b1_ref, wa_hbm, ba_ref, wb_hbm, bb_ref,
               wc_ref, bc_ref, wcls_ref, bcls_ref,
               a_out_ref, sc_out_ref, preds_ref,
               m_ref, s_ref, num_ref, w1_v, wa_v, wb_v, sem):
    i = pl.program_id(0)
    k = pl.num_programs(0)

    @pl.when(i == 0)
    def _():
        m_ref[0] = jnp.float32(_NEG)
        s_ref[0] = jnp.float32(0.0)
        num_ref[...] = jnp.zeros_like(num_ref)
        c1 = pltpu.make_async_copy(w1_hbm, w1_v, sem)
        ca = pltpu.make_async_copy(wa_hbm, wa_v, sem)
        cb = pltpu.make_async_copy(wb_hbm, wb_v, sem)
        c1.start()
        ca.start()
        cb.start()
        c1.wait()
        ca.wait()
        cb.wait()

    hf = jnp.maximum(_mm(x_ref[...], w1_v) + b1_ref[...], 0.0)
    a = jnp.tanh(_mm(hf, wa_v) + ba_ref[...])
    g = jax.nn.sigmoid(_mm(hf, wb_v) + bb_ref[...])
    att = _mm(wc_ref[...], a * g) + bc_ref[...]
    scores = _mm(wcls_ref[...], hf) + bcls_ref[...]
    pad = a_out_ref.shape[1] - att.shape[1]
    a_out_ref[...] = jnp.concatenate(
        [att, jnp.full((1, pad), _NEG, jnp.float32)], axis=1)
    sc_out_ref[...] = jnp.concatenate(
        [scores, jnp.zeros((2, pad), jnp.float32)], axis=1)

    m_old = m_ref[0]
    m_new = jnp.maximum(m_old, jnp.max(att))
    c = jnp.exp(m_old - m_new)
    e = jnp.exp(att - m_new)
    s_ref[0] = s_ref[0] * c + jnp.sum(e)
    num_ref[...] = num_ref[...] * c + jnp.sum(scores * e, axis=1,
                                              keepdims=True)
    m_ref[0] = m_new

    @pl.when(i == k - 1)
    def _():
        preds_ref[...] = num_ref[...] / s_ref[0]


def _merge_top(tv, ti, xs, ixs):
    take = xs > tv
    tv2 = jnp.where(take, xs, tv)
    ti2 = jnp.where(take, ixs, ti)
    return plsc.sort_key_val(tv2, ti2, descending=True)


def _merge_bot(bv, bi, xs, ixs):
    take = xs < bv
    bv2 = jnp.where(take, xs, bv)
    bi2 = jnp.where(take, ixs, bi)
    return plsc.sort_key_val(bv2, bi2, descending=False)


def _make_topk_gather(blk, pblk, chunk, d):
    mesh = plsc.VectorSubcoreMesh(core_axis_name="c", subcore_axis_name="s",
                                  num_cores=1)
    nc = _NS * _L

    @functools.partial(
        pl.kernel,
        out_type=jax.ShapeDtypeStruct((_L, d), jnp.float32),
        mesh=mesh,
        scratch_types=[pltpu.VMEM((chunk,), jnp.float32),
                       pltpu.VMEM((_L,), jnp.float32),
                       pltpu.VMEM((_L,), jnp.int32),
                       pltpu.VMEM_SHARED((2 * nc,), jnp.float32),
                       pltpu.VMEM_SHARED((2 * nc,), jnp.int32),
                       pltpu.VMEM((2 * nc,), jnp.float32),
                       pltpu.VMEM((2 * nc,), jnp.int32),
                       pltpu.VMEM((2 * _L,), jnp.int32),
                       pltpu.VMEM((_L,), jnp.int32),
                       pltpu.VMEM((_L, d), jnp.float32),
                       pltpu.SemaphoreType.DMA],
        compiler_params=pltpu.CompilerParams(needs_layout_passes=False),
    )
    def topk_gather(a_hbm, h_hbm, out_hbm, a_v, st_v, st_i,
                    sh_f, sh_i, cf_v, ci_v, idx_v, idx8_v, rows_v, sem):
        core = lax.axis_index("c")
        sub = lax.axis_index("s")

        @pl.when(core == 0)
        def _():
            base = sub * chunk
            pltpu.sync_copy(a_hbm.at[pl.ds(base, chunk)], a_v)
            iota = lax.iota(jnp.int32, _L)

            def body(j, carry):
                tv, ti, bv, bi = carry
                x = a_v[pl.ds(j * _L, _L)]
                p = base + j * _L + iota
                blki = p // pblk
                valid = (p - blki * pblk) < blk
                gidx = p - (pblk - blk) * blki
                xt = jnp.where(valid, x, _NEG)
                xb = jnp.where(valid, x, _POS)
                xts, its = plsc.sort_key_val(xt, gidx)
                tv, ti = _merge_top(tv, ti, xts, its)
                xbs, ibs = plsc.sort_key_val(xb, gidx, descending=True)
                bv, bi = _merge_bot(bv, bi, xbs, ibs)
                return tv, ti, bv, bi

            init = (jnp.full((_L,), _NEG, jnp.float32),
                    jnp.zeros((_L,), jnp.int32),
                    jnp.full((_L,), _POS, jnp.float32),
                    jnp.zeros((_L,), jnp.int32))
            tv, ti, bv, bi = lax.fori_loop(0, chunk // _L, body, init)

            st_v[...] = tv
            pltpu.sync_copy(st_v, sh_f.at[pl.ds(sub * _L, _L)])
            st_i[...] = ti
            pltpu.sync_copy(st_i, sh_i.at[pl.ds(sub * _L, _L)])
            st_v[...] = bv
            pltpu.sync_copy(st_v, sh_f.at[pl.ds(nc + sub * _L, _L)])
            st_i[...] = bi
            pltpu.sync_copy(st_i, sh_i.at[pl.ds(nc + sub * _L, _L)])

        plsc.subcore_barrier()

        @pl.when((core == 0) & (sub == 0))
        def _():
            pltpu.sync_copy(sh_f, cf_v)
            pltpu.sync_copy(sh_i, ci_v)

            def body_t(r, carry):
                tv, ti = carry
                xs, ixs = plsc.sort_key_val(cf_v[pl.ds(r * _L, _L)],
                                            ci_v[pl.ds(r * _L, _L)])
                return _merge_top(tv, ti, xs, ixs)

            def body_b(r, carry):
                bv, bi = carry
                xs, ixs = plsc.sort_key_val(cf_v[pl.ds(nc + r * _L, _L)],
                                            ci_v[pl.ds(nc + r * _L, _L)],
                                            descending=True)
                return _merge_bot(bv, bi, xs, ixs)

            init_t = plsc.sort_key_val(cf_v[pl.ds(0, _L)],
                                       ci_v[pl.ds(0, _L)], descending=True)
            tv, ti = lax.fori_loop(1, _NS, body_t, init_t)
            init_b = plsc.sort_key_val(cf_v[pl.ds(nc, _L)],
                                       ci_v[pl.ds(nc, _L)], descending=False)
            bv, bi = lax.fori_loop(1, _NS, body_b, init_b)

            idx_v[pl.ds(0, _L)] = ti
            idx_v[pl.ds(8, _L)] = bi
            idx8_v[...] = idx_v[pl.ds(0, _L)]
            pltpu.async_copy(h_hbm.at[idx8_v], rows_v, sem).wait()
            pltpu.sync_copy(rows_v, out_hbm)

    return topk_gather


def _loss_body(hs_ref, w1_ref, b1_ref, wi0_ref, bi0_ref, wi1_ref, bi1_ref,
               lbl_ref, loss_ref):
    hs = hs_ref[...]
    hf = jnp.maximum(_mm(hs, w1_ref) + b1_ref[...], 0.0)

    rows = lax.broadcasted_iota(jnp.int32, (_L, 1), 0)
    is_top = (rows < 8).astype(jnp.float32)

    def ce(w_ref, b_ref):
        logits = _mm(hf, w_ref) + b_ref[...]
        l0 = logits[:, 0:1]
        l1 = logits[:, 1:2]
        m = jnp.maximum(l0, l1)
        lse = m + jnp.log(jnp.exp(l0 - m) + jnp.exp(l1 - m))
        lt = is_top * l1 + (1.0 - is_top) * l0
        return jnp.sum(lse - lt) / 16.0

    loss0 = ce(wi0_ref, bi0_ref)
    loss1 = ce(wi1_ref, bi1_ref)
    wl = (lbl_ref[0] == 1).astype(jnp.float32)
    loss_ref[0] = (1.0 - wl) * loss0 + wl * loss1


def kernel(h, label, W1, b1, Wa, ba, Wb, bb, Wc, bc, Wcls, bcls,
           Wi0, bi0, Wi1, bi1):
    n = h.shape[1]
    d = h.shape[2]
    x = h.reshape(n, d)

    blk = 2000
    assert n % blk == 0
    grid = n // blk
    pblk = 2048

    b1r = b1.reshape(1, -1)
    bar = ba.reshape(1, -1)
    bbr = bb.reshape(1, -1)
    wcr = Wc.reshape(1, -1)
    bcr = bc.reshape(1, 1)
    bclsr = bcls.reshape(-1, 1)

    full = lambda s: pl.BlockSpec(s, lambda i: (0, 0))
    a_raw, scores, preds = pl.pallas_call(
        _main_body,
        grid=(grid,),
        in_specs=[
            pl.BlockSpec((blk, d), lambda i: (i, 0)),
            pl.BlockSpec(memory_space=pl.ANY),
            full((1, 512)),
            pl.BlockSpec(memory_space=pl.ANY),
            full((1, 256)),
            pl.BlockSpec(memory_space=pl.ANY),
            full((1, 256)),
            full((1, 256)), full((1, 1)),
            full((2, 512)), full((2, 1)),
        ],
        out_specs=[
            pl.BlockSpec((1, pblk), lambda i: (0, i)),
            pl.BlockSpec((2, pblk), lambda i: (0, i)),
            pl.BlockSpec((2, 1), lambda i: (0, 0)),
        ],
        out_shape=[
            jax.ShapeDtypeStruct((1, grid * pblk), jnp.float32),
            jax.ShapeDtypeStruct((2, grid * pblk), jnp.float32),
            jax.ShapeDtypeStruct((2, 1), jnp.float32),
        ],
        scratch_shapes=[
            pltpu.SMEM((1,), jnp.float32),
            pltpu.SMEM((1,), jnp.float32),
            pltpu.VMEM((2, 1), jnp.float32),
            pltpu.VMEM((512, d), jnp.float32),
            pltpu.VMEM((256, 512), jnp.float32),
            pltpu.VMEM((256, 512), jnp.float32),
            pltpu.SemaphoreType.DMA,
        ],
    )(x, W1, b1r, Wa, bar, Wb, bbr, wcr, bcr, Wcls, bclsr)

    pad_n = grid * pblk
    assert pad_n % (_NS * _L) == 0
    chunk = pad_n // _NS
    a_pad = a_raw.reshape(pad_n)

    h_sel = _make_topk_gather(blk, pblk, chunk, d)(a_pad, x)

    loss = pl.pallas_call(
        _loss_body,
        in_specs=[
            pl.BlockSpec((_L, d), lambda: (0, 0)),
            pl.BlockSpec((512, d), lambda: (0, 0)),
            pl.BlockSpec((1, 512), lambda: (0, 0)),
            pl.BlockSpec((2, 512), lambda: (0, 0)),
            pl.BlockSpec((1, 2), lambda: (0, 0)),
            pl.BlockSpec((2, 512), lambda: (0, 0)),
            pl.BlockSpec((1, 2), lambda: (0, 0)),
            pl.BlockSpec(memory_space=pltpu.SMEM),
        ],
        out_specs=pl.BlockSpec(memory_space=pltpu.SMEM),
        out_shape=jax.ShapeDtypeStruct((1,), jnp.float32),
    )(h_sel, W1, b1r, Wi0, bi0.reshape(1, -1),
      Wi1, bi1.reshape(1, -1), label)

    a_leaf = a_raw.reshape(grid, pblk)[:, :blk].reshape(n, 1)
    s_leaf = scores.reshape(2, grid, pblk)[:, :, :blk].reshape(2, n).T
    return (preds.reshape(1, 2), s_leaf, a_leaf, loss.reshape(()))


# speedup vs baseline: 1.0175x; 1.0175x over previous
import functools

import jax
import jax.numpy as jnp
from jax import lax
from jax.experimental import pallas as pl
from jax.experimental.pallas import tpu as pltpu
from jax.experimental.pallas import tpu_sc as plsc

_NS = 16
_L = 16

_NEG = -3.0e38
_POS = 3.0e38

_DN1 = (((1,), (1,)), ((), ()))


def _mm(x, w_ref):
    return lax.dot_general(x, w_ref[...], _DN1,
                           preferred_element_type=jnp.float32)


def _main_body(x_ref, w1_ref, b1_ref, wa_ref, ba_ref, wb_ref, bb_ref,
               wc_ref, bc_ref, wcls_ref, bcls_ref,
               a_out_ref, sc_out_ref, preds_ref,
               m_ref, s_ref, num_ref):
    i = pl.program_id(0)
    k = pl.num_programs(0)

    @pl.when(i == 0)
    def _():
        m_ref[0] = jnp.float32(_NEG)
        s_ref[0] = jnp.float32(0.0)
        num_ref[...] = jnp.zeros_like(num_ref)

    hf = jnp.maximum(_mm(x_ref[...], w1_ref) + b1_ref[...], 0.0)
    a = jnp.tanh(_mm(hf, wa_ref) + ba_ref[...])
    g = jax.nn.sigmoid(_mm(hf, wb_ref) + bb_ref[...])
    att = _mm(wc_ref[...], a * g) + bc_ref[...]
    scores = _mm(wcls_ref[...], hf) + bcls_ref[...]
    pad = a_out_ref.shape[1] - att.shape[1]
    a_out_ref[...] = jnp.concatenate(
        [att, jnp.full((1, pad), _NEG, jnp.float32)], axis=1)
    sc_out_ref[...] = jnp.concatenate(
        [scores, jnp.zeros((2, pad), jnp.float32)], axis=1)

    m_old = m_ref[0]
    m_new = jnp.maximum(m_old, jnp.max(att))
    c = jnp.exp(m_old - m_new)
    e = jnp.exp(att - m_new)
    s_ref[0] = s_ref[0] * c + jnp.sum(e)
    num_ref[...] = num_ref[...] * c + jnp.sum(scores * e, axis=1,
                                              keepdims=True)
    m_ref[0] = m_new

    @pl.when(i == k - 1)
    def _():
        preds_ref[...] = num_ref[...] / s_ref[0]


def _merge_top(tv, ti, xs, ixs):
    take = xs > tv
    tv2 = jnp.where(take, xs, tv)
    ti2 = jnp.where(take, ixs, ti)
    return plsc.sort_key_val(tv2, ti2, descending=True)


def _merge_bot(bv, bi, xs, ixs):
    take = xs < bv
    bv2 = jnp.where(take, xs, bv)
    bi2 = jnp.where(take, ixs, bi)
    return plsc.sort_key_val(bv2, bi2, descending=False)


def _make_topk_gather(blk, pblk, chunk, d):
    mesh = plsc.VectorSubcoreMesh(core_axis_name="c", subcore_axis_name="s",
                                  num_cores=1)
    nc = _NS * _L

    @functools.partial(
        pl.kernel,
        out_type=jax.ShapeDtypeStruct((_L, d), jnp.float32),
        mesh=mesh,
        scratch_types=[pltpu.VMEM((chunk,), jnp.float32),
                       pltpu.VMEM((_L,), jnp.float32),
                       pltpu.VMEM((_L,), jnp.int32),
                       pltpu.VMEM_SHARED((2 * nc,), jnp.float32),
                       pltpu.VMEM_SHARED((2 * nc,), jnp.int32),
                       pltpu.VMEM((2 * nc,), jnp.float32),
                       pltpu.VMEM((2 * nc,), jnp.int32),
                       pltpu.VMEM((2 * _L,), jnp.int32),
                       pltpu.VMEM((_L,), jnp.int32),
                       pltpu.VMEM((_L, d), jnp.float32),
                       pltpu.SemaphoreType.DMA],
        compiler_params=pltpu.CompilerParams(needs_layout_passes=False),
    )
    def topk_gather(a_hbm, h_hbm, out_hbm, a_v, st_v, st_i,
                    sh_f, sh_i, cf_v, ci_v, idx_v, idx8_v, rows_v, sem):
        core = lax.axis_index("c")
        sub = lax.axis_index("s")

        @pl.when(core == 0)
        def _():
            base = sub * chunk
            pltpu.sync_copy(a_hbm.at[pl.ds(base, chunk)], a_v)
            iota = lax.iota(jnp.int32, _L)

            def body(j, carry):
                tv, ti, bv, bi = carry
                x = a_v[pl.ds(j * _L, _L)]
                p = base + j * _L + iota
                blki = p // pblk
                valid = (p - blki * pblk) < blk
                gidx = p - (pblk - blk) * blki
                xt = jnp.where(valid, x, _NEG)
                xb = jnp.where(valid, x, _POS)
                xts, its = plsc.sort_key_val(xt, gidx)
                tv, ti = _merge_top(tv, ti, xts, its)
                xbs, ibs = plsc.sort_key_val(xb, gidx, descending=True)
                bv, bi = _merge_bot(bv, bi, xbs, ibs)
                return tv, ti, bv, bi

            init = (jnp.full((_L,), _NEG, jnp.float32),
                    jnp.zeros((_L,), jnp.int32),
                    jnp.full((_L,), _POS, jnp.float32),
                    jnp.zeros((_L,), jnp.int32))
            tv, ti, bv, bi = lax.fori_loop(0, chunk // _L, body, init)

            st_v[...] = tv
            pltpu.sync_copy(st_v, sh_f.at[pl.ds(sub * _L, _L)])
            st_i[...] = ti
            pltpu.sync_copy(st_i, sh_i.at[pl.ds(sub * _L, _L)])
            st_v[...] = bv
            pltpu.sync_copy(st_v, sh_f.at[pl.ds(nc + sub * _L, _L)])
            st_i[...] = bi
            pltpu.sync_copy(st_i, sh_i.at[pl.ds(nc + sub * _L, _L)])

        plsc.subcore_barrier()

        @pl.when((core == 0) & (sub == 0))
        def _():
            pltpu.sync_copy(sh_f, cf_v)
            pltpu.sync_copy(sh_i, ci_v)

            def body_t(r, carry):
                tv, ti = carry
                xs, ixs = plsc.sort_key_val(cf_v[pl.ds(r * _L, _L)],
                                            ci_v[pl.ds(r * _L, _L)])
                return _merge_top(tv, ti, xs, ixs)

            def body_b(r, carry):
                bv, bi = carry
                xs, ixs = plsc.sort_key_val(cf_v[pl.ds(nc + r * _L, _L)],
                                            ci_v[pl.ds(nc + r * _L, _L)],
                                            descending=True)
                return _merge_bot(bv, bi, xs, ixs)

            init_t = plsc.sort_key_val(cf_v[pl.ds(0, _L)],
                                       ci_v[pl.ds(0, _L)], descending=True)
            tv, ti = lax.fori_loop(1, _NS, body_t, init_t)
            init_b = plsc.sort_key_val(cf_v[pl.ds(nc, _L)],
                                       ci_v[pl.ds(nc, _L)], descending=False)
            bv, bi = lax.fori_loop(1, _NS, body_b, init_b)

            idx_v[pl.ds(0, _L)] = ti
            idx_v[pl.ds(8, _L)] = bi
            idx8_v[...] = idx_v[pl.ds(0, _L)]
            pltpu.async_copy(h_hbm.at[idx8_v], rows_v, sem).wait()
            pltpu.sync_copy(rows_v, out_hbm)

    return topk_gather


def _loss_body(hs_ref, w1_ref, b1_ref, wi0_ref, bi0_ref, wi1_ref, bi1_ref,
               lbl_ref, loss_ref):
    hs = hs_ref[...]
    hf = jnp.maximum(_mm(hs, w1_ref) + b1_ref[...], 0.0)

    rows = lax.broadcasted_iota(jnp.int32, (_L, 1), 0)
    is_top = (rows < 8).astype(jnp.float32)

    def ce(w_ref, b_ref):
        logits = _mm(hf, w_ref) + b_ref[...]
        l0 = logits[:, 0:1]
        l1 = logits[:, 1:2]
        m = jnp.maximum(l0, l1)
        lse = m + jnp.log(jnp.exp(l0 - m) + jnp.exp(l1 - m))
        lt = is_top * l1 + (1.0 - is_top) * l0
        return jnp.sum(lse - lt) / 16.0

    loss0 = ce(wi0_ref, bi0_ref)
    loss1 = ce(wi1_ref, bi1_ref)
    wl = (lbl_ref[0] == 1).astype(jnp.float32)
    loss_ref[0] = (1.0 - wl) * loss0 + wl * loss1


def kernel(h, label, W1, b1, Wa, ba, Wb, bb, Wc, bc, Wcls, bcls,
           Wi0, bi0, Wi1, bi1):
    n = h.shape[1]
    d = h.shape[2]
    x = h.reshape(n, d)

    blk = 2000
    assert n % blk == 0
    grid = n // blk
    pblk = 2048

    b1r = b1.reshape(1, -1)
    bar = ba.reshape(1, -1)
    bbr = bb.reshape(1, -1)
    wcr = Wc.reshape(1, -1)
    bcr = bc.reshape(1, 1)
    bclsr = bcls.reshape(-1, 1)

    full = lambda s: pl.BlockSpec(s, lambda i: (0, 0))
    a_raw, scores, preds = pl.pallas_call(
        _main_body,
        grid=(grid,),
        in_specs=[
            pl.BlockSpec((blk, d), lambda i: (i, 0)),
            full((512, d)), full((1, 512)),
            full((256, 512)), full((1, 256)),
            full((256, 512)), full((1, 256)),
            full((1, 256)), full((1, 1)),
            full((2, 512)), full((2, 1)),
        ],
        out_specs=[
            pl.BlockSpec((1, pblk), lambda i: (0, i)),
            pl.BlockSpec((2, pblk), lambda i: (0, i)),
            pl.BlockSpec((2, 1), lambda i: (0, 0)),
        ],
        out_shape=[
            jax.ShapeDtypeStruct((1, grid * pblk), jnp.float32),
            jax.ShapeDtypeStruct((2, grid * pblk), jnp.float32),
            jax.ShapeDtypeStruct((2, 1), jnp.float32),
        ],
        scratch_shapes=[
            pltpu.SMEM((1,), jnp.float32),
            pltpu.SMEM((1,), jnp.float32),
            pltpu.VMEM((2, 1), jnp.float32),
        ],
    )(x, W1, b1r, Wa, bar, Wb, bbr, wcr, bcr, Wcls, bclsr)

    pad_n = grid * pblk
    assert pad_n % (_NS * _L) == 0
    chunk = pad_n // _NS
    a_pad = a_raw.reshape(pad_n)

    h_sel = _make_topk_gather(blk, pblk, chunk, d)(a_pad, x)

    loss = pl.pallas_call(
        _loss_body,
        in_specs=[
            pl.BlockSpec((_L, d), lambda: (0, 0)),
            pl.BlockSpec((512, d), lambda: (0, 0)),
            pl.BlockSpec((1, 512), lambda: (0, 0)),
            pl.BlockSpec((2, 512), lambda: (0, 0)),
            pl.BlockSpec((1, 2), lambda: (0, 0)),
            pl.BlockSpec((2, 512), lambda: (0, 0)),
            pl.BlockSpec((1, 2), lambda: (0, 0)),
            pl.BlockSpec(memory_space=pltpu.SMEM),
        ],
        out_specs=pl.BlockSpec(memory_space=pltpu.SMEM),
        out_shape=jax.ShapeDtypeStruct((1,), jnp.float32),
    )(h_sel, W1, b1r, Wi0, bi0.reshape(1, -1),
      Wi1, bi1.reshape(1, -1), label)

    a_leaf = a_raw.reshape(grid, pblk)[:, :blk].reshape(n, 1)
    s_leaf = scores.reshape(2, grid, pblk)[:, :, :blk].reshape(2, n).T
    return (preds.reshape(1, 2), s_leaf, a_leaf, loss.reshape(()))
